# Initial kernel scaffold; baseline (speedup 1.0000x reference)
import jax
import jax.numpy as jnp
from jax.experimental import pallas as pl


def kernel(x, router_w, shared_wg, shared_wu, shared_wd, routed_wg, routed_wu, routed_wd):
    raise NotImplementedError("write your pallas kernel here")


# speedup vs baseline: 1.4907x; 1.4907x over previous
import functools

import jax
import jax.numpy as jnp
from jax.experimental import pallas as pl
from jax.experimental.pallas import tpu as pltpu

B, SEQ, D = 32, 1, 1024
E_ROUTED, E_SHARED, H, TOP_K = 64, 2, 512, 2
T = B * SEQ


def _routing_body(x_ref, rw_ref, logits_ref, gates_ref, idx_ref, gdense_ref):
    x = x_ref[...]
    logits = jnp.dot(x, rw_ref[...], preferred_element_type=jnp.float32)
    logits_ref[...] = logits
    m = jnp.max(logits, axis=1, keepdims=True)
    e = jnp.exp(logits - m)
    p = e / jnp.sum(e, axis=1, keepdims=True)
    iota = jax.lax.broadcasted_iota(jnp.int32, (T, E_ROUTED), 1)
    m1 = jnp.max(p, axis=1, keepdims=True)
    i1 = jnp.min(jnp.where(p == m1, iota, E_ROUTED), axis=1, keepdims=True)
    sel1 = iota == i1
    p2 = jnp.where(sel1, -1.0, p)
    m2 = jnp.max(p2, axis=1, keepdims=True)
    i2 = jnp.min(jnp.where(p2 == m2, iota, E_ROUTED), axis=1, keepdims=True)
    sel2 = iota == i2
    gdense_ref[...] = jnp.where(sel1, m1, 0.0) + jnp.where(sel2, m2, 0.0)
    gates_ref[...] = jnp.concatenate([m1, m2], axis=1)
    idx_ref[...] = jnp.concatenate([i1, i2], axis=1)


def _moe_body(sched_ref, nact_ref, x_ref, g_ref,
              swg_ref, swu_ref, swd_ref, rwg_ref, rwu_ref, rwd_ref, out_ref):
    i = pl.program_id(0)
    xb = x_ref[...].astype(jnp.bfloat16)

    @pl.when(i == 0)
    def _shared():
        acc = jnp.zeros((T, D), jnp.float32)
        for e in range(E_SHARED):
            hg = jnp.dot(xb, swg_ref[e].astype(jnp.bfloat16),
                         preferred_element_type=jnp.float32)
            hu = jnp.dot(xb, swu_ref[e].astype(jnp.bfloat16),
                         preferred_element_type=jnp.float32)
            h = hg * jax.lax.logistic(hg) * hu
            acc = acc + jnp.dot(h.astype(jnp.bfloat16),
                                swd_ref[e].astype(jnp.bfloat16),
                                preferred_element_type=jnp.float32)
        out_ref[...] = acc / float(E_SHARED)

    @pl.when(i < nact_ref[0])
    def _routed():
        eid = sched_ref[i]
        hg = jnp.dot(xb, rwg_ref[0].astype(jnp.bfloat16),
                     preferred_element_type=jnp.float32)
        hu = jnp.dot(xb, rwu_ref[0].astype(jnp.bfloat16),
                     preferred_element_type=jnp.float32)
        h = hg * jax.lax.logistic(hg) * hu
        o = jnp.dot(h.astype(jnp.bfloat16), rwd_ref[0].astype(jnp.bfloat16),
                    preferred_element_type=jnp.float32)
        iota = jax.lax.broadcasted_iota(jnp.int32, (T, E_ROUTED), 1)
        scale = jnp.sum(jnp.where(iota == eid, g_ref[...], 0.0),
                        axis=1, keepdims=True)
        out_ref[...] = out_ref[...] + o * scale


@functools.partial(jax.jit, static_argnames=("interpret",))
def kernel(x, router_w, shared_wg, shared_wu, shared_wd,
           routed_wg, routed_wu, routed_wd, interpret=False):
    xf = x.reshape(T, D)

    logits, gates, indices, g_dense = pl.pallas_call(
        _routing_body,
        out_shape=(
            jax.ShapeDtypeStruct((T, E_ROUTED), jnp.float32),
            jax.ShapeDtypeStruct((T, TOP_K), jnp.float32),
            jax.ShapeDtypeStruct((T, TOP_K), jnp.int32),
            jax.ShapeDtypeStruct((T, E_ROUTED), jnp.float32),
        ),
        interpret=interpret,
    )(xf, router_w)

    ar = jnp.arange(E_ROUTED, dtype=jnp.int32)
    active = (indices.reshape(-1)[:, None] == ar[None, :]).any(axis=0)
    order = jnp.argsort(jnp.where(active, ar, ar + E_ROUTED)).astype(jnp.int32)
    nact = jnp.sum(active).astype(jnp.int32)
    last = order[nact - 1]
    sched = jnp.where(ar < nact, order, last)

    out = pl.pallas_call(
        _moe_body,
        grid_spec=pltpu.PrefetchScalarGridSpec(
            num_scalar_prefetch=2,
            grid=(E_ROUTED,),
            in_specs=[
                pl.BlockSpec((T, D), lambda i, sched, nact: (0, 0)),
                pl.BlockSpec((T, E_ROUTED), lambda i, sched, nact: (0, 0)),
                pl.BlockSpec((E_SHARED, D, H), lambda i, sched, nact: (0, 0, 0)),
                pl.BlockSpec((E_SHARED, D, H), lambda i, sched, nact: (0, 0, 0)),
                pl.BlockSpec((E_SHARED, H, D), lambda i, sched, nact: (0, 0, 0)),
                pl.BlockSpec((1, D, H), lambda i, sched, nact: (sched[i], 0, 0)),
                pl.BlockSpec((1, D, H), lambda i, sched, nact: (sched[i], 0, 0)),
                pl.BlockSpec((1, H, D), lambda i, sched, nact: (sched[i], 0, 0)),
            ],
            out_specs=pl.BlockSpec((T, D), lambda i, sched, nact: (0, 0)),
        ),
        out_shape=jax.ShapeDtypeStruct((T, D), jnp.float32),
        compiler_params=pltpu.CompilerParams(
            dimension_semantics=("arbitrary",),
        ),
        interpret=interpret,
    )(sched, nact.reshape(1), xf, g_dense,
      shared_wg, shared_wu, shared_wd, routed_wg, routed_wu, routed_wd)

    return out.reshape(B, SEQ, D), logits, indices, gates
